# Initial kernel scaffold; baseline (speedup 1.0000x reference)
import jax
import jax.numpy as jnp
from jax.experimental import pallas as pl


def kernel(x, rbf, sbf, idx_kj, idx_ji, bt, lambda_d, alpha, W_rbf1, W_rbf2, W_sbf1, W_sbf2, W_kj, b_kj, W_ji, b_ji, W_down, W_up, W_b1, b_b1, W_b2, b_b2, W_lin, b_lin, W_a1, b_a1, W_a2, b_a2):
    raise NotImplementedError("write your pallas kernel here")


# speedup vs baseline: 2.0953x; 2.0953x over previous
import functools

import jax
import jax.numpy as jnp
from jax import lax
from jax.experimental import pallas as pl
from jax.experimental.pallas import tpu as pltpu
from jax.experimental.pallas import tpu_sc as plsc


def _silu(v):
    return v * jax.nn.sigmoid(v)


def _dot(a, b):
    return jnp.dot(a, b, preferred_element_type=jnp.float32)


def _edge_body(x_ref, rbf_ref, wkj_ref, bkj_ref, wr1_ref, wr2_ref, wdn_ref,
               tmp_ref):
    x = x_ref[...]
    t1 = _silu(_dot(x, wkj_ref[...]) + bkj_ref[...])
    r = _dot(_dot(rbf_ref[...], wr1_ref[...]), wr2_ref[...])
    v = _silu(_dot(t1 * r, wdn_ref[...]))
    tmp_ref[...] = jnp.concatenate([v, jnp.zeros_like(v)], axis=1)


def _edge_mlp(x, rbf, wkj, bkj, wr1, wr2, wdn, blk):
    e, h = x.shape
    nr = rbf.shape[1]
    be = wr1.shape[1]
    ie = wdn.shape[1]
    return pl.pallas_call(
        _edge_body,
        grid=(e // blk,),
        in_specs=[
            pl.BlockSpec((blk, h), lambda i: (i, 0)),
            pl.BlockSpec((blk, nr), lambda i: (i, 0)),
            pl.BlockSpec((h, h), lambda i: (0, 0)),
            pl.BlockSpec((1, h), lambda i: (0, 0)),
            pl.BlockSpec((nr, be), lambda i: (0, 0)),
            pl.BlockSpec((be, h), lambda i: (0, 0)),
            pl.BlockSpec((h, ie), lambda i: (0, 0)),
        ],
        out_specs=pl.BlockSpec((blk, 2 * ie), lambda i: (i, 0)),
        out_shape=jax.ShapeDtypeStruct((e, 2 * ie), jnp.float32),
        compiler_params=pltpu.CompilerParams(
            dimension_semantics=("arbitrary",)),
    )(x, rbf, wkj, bkj, wr1, wr2, wdn)


def _sbf_body(sbf_ref, ws1_ref, ws2_ref, out_ref):
    v = _dot(_dot(sbf_ref[...], ws1_ref[...]), ws2_ref[...])
    out_ref[...] = jnp.concatenate([v, jnp.zeros_like(v)], axis=1)


def _sbf_mlp(sbf, ws1, ws2, blk):
    t, sr = sbf.shape
    be = ws1.shape[1]
    ie = ws2.shape[1]
    return pl.pallas_call(
        _sbf_body,
        grid=(t // blk,),
        in_specs=[
            pl.BlockSpec((blk, sr), lambda i: (i, 0)),
            pl.BlockSpec((sr, be), lambda i: (0, 0)),
            pl.BlockSpec((be, ie), lambda i: (0, 0)),
        ],
        out_specs=pl.BlockSpec((blk, 2 * ie), lambda i: (i, 0)),
        out_shape=jax.ShapeDtypeStruct((t, 2 * ie), jnp.float32),
        compiler_params=pltpu.CompilerParams(
            dimension_semantics=("arbitrary",)),
    )(sbf, ws1, ws2)


def _tail_body(xt_ref, x_ref, wup_ref, wji_ref, bji_ref, wb1_ref, bb1_ref,
               wb2_ref, bb2_ref, wl_ref, bl_ref, wa1_ref, ba1_ref, wa2_ref,
               ba2_ref, out_ref):
    x = x_ref[...]
    x_ji = _silu(_dot(x, wji_ref[...]) + bji_ref[...])
    x_kj = _silu(_dot(xt_ref[...], wup_ref[...]))
    h = x_ji + x_kj
    h = h + _silu(_dot(_silu(_dot(h, wb1_ref[...]) + bb1_ref[...]),
                       wb2_ref[...]) + bb2_ref[...])
    h = _silu(_dot(h, wl_ref[...]) + bl_ref[...]) + x
    h = h + _silu(_dot(_silu(_dot(h, wa1_ref[...]) + ba1_ref[...]),
                       wa2_ref[...]) + ba2_ref[...])
    out_ref[...] = h


def _tail_mlp(xt, x, wup, wji, bji, wb1, bb1, wb2, bb2, wl, bl, wa1, ba1,
              wa2, ba2, blk):
    e, h = x.shape
    ie = xt.shape[1]
    mat = lambda a, b: pl.BlockSpec((a, b), lambda i: (0, 0))
    return pl.pallas_call(
        _tail_body,
        grid=(e // blk,),
        in_specs=[
            pl.BlockSpec((blk, ie), lambda i: (i, 0)),
            pl.BlockSpec((blk, h), lambda i: (i, 0)),
            mat(ie, h), mat(h, h), mat(1, h), mat(h, h), mat(1, h),
            mat(h, h), mat(1, h), mat(h, h), mat(1, h), mat(h, h),
            mat(1, h), mat(h, h), mat(1, h),
        ],
        out_specs=pl.BlockSpec((blk, h), lambda i: (i, 0)),
        out_shape=jax.ShapeDtypeStruct((e, h), jnp.float32),
        compiler_params=pltpu.CompilerParams(
            dimension_semantics=("arbitrary",)),
    )(xt, x, wup, wji, bji, wb1, bb1, wb2, bb2, wl, bl, wa1, ba1, wa2, ba2)


_BB = 64
_WSZ = 4000


def _zero_rows(rows_a, dst, off, n):
    z = 0
    while z < n:
        step = min(_BB, n - z)
        src = rows_a if step == _BB else rows_a.at[pl.ds(0, step)]
        pltpu.sync_copy(src, dst.at[pl.ds(off + z, step)])
        z += step


def _sc_segsum(tmp, sbft, idx_kj, idx_ji):
    e, ie = tmp.shape
    t = sbft.shape[0]
    n_sc = 2
    n_sub = 16
    tpt = t // n_sub
    nchunk = 10
    ch = e // (n_sc * nchunk)
    lcap = tpt + _BB
    nwin = tpt // _WSZ
    wgrp = _WSZ // 16
    share = (ch // n_sub) // 8 * 8
    zrem = (ch + 16) - (n_sub - 1) * share
    wrem = ch - (n_sub - 1) * share

    mesh = plsc.VectorSubcoreMesh(core_axis_name="c", subcore_axis_name="s")

    @functools.partial(
        pl.kernel,
        out_type=jax.ShapeDtypeStruct((e, ie), jnp.float32),
        mesh=mesh,
        compiler_params=pltpu.CompilerParams(needs_layout_passes=False),
        scratch_types=[
            pltpu.VMEM((_WSZ,), jnp.int32),
            pltpu.VMEM((_WSZ,), jnp.int32),
            pltpu.VMEM((lcap,), jnp.int32),
            pltpu.VMEM((lcap,), jnp.int32),
            pltpu.VMEM((_BB,), jnp.int32),
            pltpu.VMEM((_BB,), jnp.int32),
            pltpu.VMEM((_BB,), jnp.int32),
            pltpu.VMEM((_BB, ie), jnp.float32),
            pltpu.VMEM((_BB, ie), jnp.float32),
            pltpu.VMEM_SHARED((ch + 16, ie), jnp.float32),
            pltpu.SemaphoreType.DMA,
            pltpu.SemaphoreType.DMA,
        ],
    )
    def k(tmp_hbm, sbft_hbm, kj_hbm, ji_hbm, out_hbm, win_ji, win_kj,
          lst_td, lst_kj, bat_kj, bat_t, bat_dst, rows_a, rows_b, acc,
          sem1, sem2):
        c = lax.axis_index("c")
        s = lax.axis_index("s")
        iota16 = lax.iota(jnp.int32, 16)
        tbase = s * tpt
        zero16 = jnp.zeros((16,), jnp.float32)

        for chunk in range(nchunk):
            base = c * (nchunk * ch) + chunk * ch

            def zr(i, _):
                for cc in range(ie // 16):
                    rows_a[i, pl.ds(cc * 16, 16)] = zero16
                return 0

            lax.fori_loop(0, _BB, zr, 0)
            _zero_rows(rows_a, acc, s * share, share)

            @pl.when(s == n_sub - 1)
            def _zero_tail():
                _zero_rows(rows_a, acc, n_sub * share, zrem - share)

            plsc.subcore_barrier()

            m = jnp.int32(0)
            for w in range(nwin):
                woff = pl.multiple_of(tbase + w * _WSZ, 8)
                pltpu.sync_copy(ji_hbm.at[pl.ds(woff, _WSZ)], win_ji)
                pltpu.sync_copy(kj_hbm.at[pl.ds(woff, _WSZ)], win_kj)

                def grp(g, mm, w=w):
                    ji = win_ji[pl.ds(g * 16, 16)]
                    msk = (ji >= base) & (ji < base + ch)
                    mi = jnp.where(msk, 1, 0)
                    incl = plsc.cumsum(mi)
                    pos = mm + incl - mi
                    td = ((ji - base) << 15) | (w * _WSZ + g * 16 + iota16)
                    plsc.store_scatter(lst_td, [pos], td, mask=msk)
                    kjv = win_kj[pl.ds(g * 16, 16)]
                    plsc.store_scatter(lst_kj, [pos], kjv, mask=msk)
                    return mm + incl[15]

                m = lax.fori_loop(0, wgrp, grp, m)

            nb = (m + (_BB - 1)) // _BB

            def batch(b, _):
                o = b * _BB
                for g in range(_BB // 16):
                    td = lst_td[pl.ds(o + g * 16, 16)]
                    kj = lst_kj[pl.ds(o + g * 16, 16)]
                    pos = o + g * 16 + iota16
                    pred = pos < m
                    trel = td & jnp.int32(0x7FFF)
                    dstv = lax.shift_right_logical(td, 15)
                    bat_t[pl.ds(g * 16, 16)] = jnp.where(
                        pred, trel + tbase, tbase)
                    bat_kj[pl.ds(g * 16, 16)] = jnp.where(pred, kj, iota16)
                    bat_dst[pl.ds(g * 16, 16)] = jnp.where(
                        pred, dstv, jnp.int32(ch))
                cp1 = pltpu.async_copy(tmp_hbm.at[bat_kj], rows_a, sem1)
                cp2 = pltpu.async_copy(sbft_hbm.at[bat_t], rows_b, sem2)
                cp1.wait()
                cp2.wait()

                def mulrow(r, _):
                    for cc in range(ie // 32):
                        sl = pl.ds(cc * 16, 16)
                        rows_a[r, sl] = rows_a[r, sl] * rows_b[r, sl]
                    return 0

                lax.fori_loop(0, _BB, mulrow, 0)
                pltpu.sync_copy(rows_a, acc.at[bat_dst], add=True)
                return 0

            lax.fori_loop(0, nb, batch, 0)
            plsc.subcore_barrier()

            pltpu.sync_copy(acc.at[pl.ds(s * share, share)],
                            out_hbm.at[pl.ds(base + s * share, share)])

            if wrem > share:
                @pl.when(s == n_sub - 1)
                def _write_tail():
                    tail = wrem - share
                    off = n_sub * share
                    pltpu.sync_copy(acc.at[pl.ds(off, tail)],
                                    out_hbm.at[pl.ds(base + off, tail)])

            plsc.subcore_barrier()

    return k(tmp, sbft, idx_kj, idx_ji)


def kernel(x, rbf, sbf, idx_kj, idx_ji, bt, lambda_d, alpha,
           W_rbf1, W_rbf2, W_sbf1, W_sbf2, W_kj, b_kj, W_ji, b_ji,
           W_down, W_up, W_b1, b_b1, W_b2, b_b2, W_lin, b_lin,
           W_a1, b_a1, W_a2, b_a2):
    nbt = W_kj.shape[0]
    b = nbt - 1
    h = x.shape[1]
    blk = 1600

    tmp = _edge_mlp(x, rbf, W_kj[b], b_kj[b].reshape(1, h),
                    W_rbf1[b], W_rbf2[b], W_down[b], blk)
    sbft = _sbf_mlp(sbf, W_sbf1[b], W_sbf2[b], blk)
    xkt = _sc_segsum(tmp, sbft, idx_kj.astype(jnp.int32),
                     idx_ji.astype(jnp.int32))
    w_up_eff = jnp.concatenate(
        [W_up * jnp.asarray(alpha, jnp.float32), jnp.zeros_like(W_up)],
        axis=0)
    return _tail_mlp(xkt, x, w_up_eff, W_ji, b_ji.reshape(1, h),
                     W_b1, b_b1.reshape(1, h), W_b2, b_b2.reshape(1, h),
                     W_lin, b_lin.reshape(1, h), W_a1, b_a1.reshape(1, h),
                     W_a2, b_a2.reshape(1, h), blk)


# speedup vs baseline: 2.3749x; 1.1335x over previous
import functools

import jax
import jax.numpy as jnp
from jax import lax
from jax.experimental import pallas as pl
from jax.experimental.pallas import tpu as pltpu
from jax.experimental.pallas import tpu_sc as plsc


def _silu(v):
    return v * jax.nn.sigmoid(v)


def _dot(a, b):
    return jnp.dot(a, b, preferred_element_type=jnp.float32)


def _edge_body(x_ref, rbf_ref, wkj_ref, bkj_ref, wr1_ref, wr2_ref, wdn_ref,
               tmp_ref):
    x = x_ref[...]
    t1 = _silu(_dot(x, wkj_ref[...]) + bkj_ref[...])
    r = _dot(_dot(rbf_ref[...], wr1_ref[...]), wr2_ref[...])
    v = _silu(_dot(t1 * r, wdn_ref[...]))
    tmp_ref[...] = jnp.concatenate([v, jnp.zeros_like(v)], axis=1)


def _edge_mlp(x, rbf, wkj, bkj, wr1, wr2, wdn, blk):
    e, h = x.shape
    nr = rbf.shape[1]
    be = wr1.shape[1]
    ie = wdn.shape[1]
    return pl.pallas_call(
        _edge_body,
        grid=(e // blk,),
        in_specs=[
            pl.BlockSpec((blk, h), lambda i: (i, 0)),
            pl.BlockSpec((blk, nr), lambda i: (i, 0)),
            pl.BlockSpec((h, h), lambda i: (0, 0)),
            pl.BlockSpec((1, h), lambda i: (0, 0)),
            pl.BlockSpec((nr, be), lambda i: (0, 0)),
            pl.BlockSpec((be, h), lambda i: (0, 0)),
            pl.BlockSpec((h, ie), lambda i: (0, 0)),
        ],
        out_specs=pl.BlockSpec((blk, 2 * ie), lambda i: (i, 0)),
        out_shape=jax.ShapeDtypeStruct((e, 2 * ie), jnp.float32),
        compiler_params=pltpu.CompilerParams(
            dimension_semantics=("arbitrary",)),
    )(x, rbf, wkj, bkj, wr1, wr2, wdn)


def _sbf_body(sbf_ref, ws1_ref, ws2_ref, out_ref):
    v = _dot(_dot(sbf_ref[...], ws1_ref[...]), ws2_ref[...])
    out_ref[...] = jnp.concatenate([v, jnp.zeros_like(v)], axis=1)


def _sbf_mlp(sbf, ws1, ws2, blk):
    t, sr = sbf.shape
    be = ws1.shape[1]
    ie = ws2.shape[1]
    return pl.pallas_call(
        _sbf_body,
        grid=(t // blk,),
        in_specs=[
            pl.BlockSpec((blk, sr), lambda i: (i, 0)),
            pl.BlockSpec((sr, be), lambda i: (0, 0)),
            pl.BlockSpec((be, ie), lambda i: (0, 0)),
        ],
        out_specs=pl.BlockSpec((blk, 2 * ie), lambda i: (i, 0)),
        out_shape=jax.ShapeDtypeStruct((t, 2 * ie), jnp.float32),
        compiler_params=pltpu.CompilerParams(
            dimension_semantics=("arbitrary",)),
    )(sbf, ws1, ws2)


def _tail_body(xt_ref, x_ref, wup_ref, wji_ref, bji_ref, wb1_ref, bb1_ref,
               wb2_ref, bb2_ref, wl_ref, bl_ref, wa1_ref, ba1_ref, wa2_ref,
               ba2_ref, out_ref):
    x = x_ref[...]
    x_ji = _silu(_dot(x, wji_ref[...]) + bji_ref[...])
    x_kj = _silu(_dot(xt_ref[...], wup_ref[...]))
    h = x_ji + x_kj
    h = h + _silu(_dot(_silu(_dot(h, wb1_ref[...]) + bb1_ref[...]),
                       wb2_ref[...]) + bb2_ref[...])
    h = _silu(_dot(h, wl_ref[...]) + bl_ref[...]) + x
    h = h + _silu(_dot(_silu(_dot(h, wa1_ref[...]) + ba1_ref[...]),
                       wa2_ref[...]) + ba2_ref[...])
    out_ref[...] = h


def _tail_mlp(xt, x, wup, wji, bji, wb1, bb1, wb2, bb2, wl, bl, wa1, ba1,
              wa2, ba2, blk):
    e, h = x.shape
    ie = xt.shape[1]
    mat = lambda a, b: pl.BlockSpec((a, b), lambda i: (0, 0))
    return pl.pallas_call(
        _tail_body,
        grid=(e // blk,),
        in_specs=[
            pl.BlockSpec((blk, ie), lambda i: (i, 0)),
            pl.BlockSpec((blk, h), lambda i: (i, 0)),
            mat(ie, h), mat(h, h), mat(1, h), mat(h, h), mat(1, h),
            mat(h, h), mat(1, h), mat(h, h), mat(1, h), mat(h, h),
            mat(1, h), mat(h, h), mat(1, h),
        ],
        out_specs=pl.BlockSpec((blk, h), lambda i: (i, 0)),
        out_shape=jax.ShapeDtypeStruct((e, h), jnp.float32),
        compiler_params=pltpu.CompilerParams(
            dimension_semantics=("arbitrary",)),
    )(xt, x, wup, wji, bji, wb1, bb1, wb2, bb2, wl, bl, wa1, ba1, wa2, ba2)


_BB = 32
_WSZ = 4000


def _zero_rows(rows_a, dst, off, n):
    z = 0
    while z < n:
        step = min(_BB, n - z)
        src = rows_a if step == _BB else rows_a.at[pl.ds(0, step)]
        pltpu.sync_copy(src, dst.at[pl.ds(off + z, step)])
        z += step


def _sc_segsum(tmp, sbft, idx_kj, idx_ji):
    e, ie = tmp.shape
    t = sbft.shape[0]
    n_sc = 2
    n_sub = 16
    tpt = t // n_sub
    nchunk = 10
    ch = e // (n_sc * nchunk)
    lcap = tpt + 2 * _BB
    nwin = tpt // _WSZ
    wgrp = _WSZ // 16
    share = (ch // n_sub) // 8 * 8
    zrem = (ch + 16) - (n_sub - 1) * share
    wrem = ch - (n_sub - 1) * share

    mesh = plsc.VectorSubcoreMesh(core_axis_name="c", subcore_axis_name="s")

    @functools.partial(
        pl.kernel,
        out_type=jax.ShapeDtypeStruct((e, ie), jnp.float32),
        mesh=mesh,
        compiler_params=pltpu.CompilerParams(needs_layout_passes=False),
        scratch_types=[
            pltpu.VMEM((_WSZ,), jnp.int32),
            pltpu.VMEM((_WSZ,), jnp.int32),
            pltpu.VMEM((lcap,), jnp.int32),
            pltpu.VMEM((lcap,), jnp.int32),
            pltpu.VMEM((2, _BB), jnp.int32),
            pltpu.VMEM((2, _BB), jnp.int32),
            pltpu.VMEM((2, _BB), jnp.int32),
            pltpu.VMEM((_BB, ie), jnp.float32),
            pltpu.VMEM((_BB, ie), jnp.float32),
            pltpu.VMEM((_BB, ie), jnp.float32),
            pltpu.VMEM((_BB, ie), jnp.float32),
            pltpu.VMEM_SHARED((ch + 16, ie), jnp.float32),
            pltpu.SemaphoreType.DMA,
            pltpu.SemaphoreType.DMA,
            pltpu.SemaphoreType.DMA,
            pltpu.SemaphoreType.DMA,
        ],
    )
    def k(tmp_hbm, sbft_hbm, kj_hbm, ji_hbm, out_hbm, win_ji, win_kj,
          lst_td, lst_kj, bat_kj, bat_t, bat_dst, rows_a0, rows_b0,
          rows_a1, rows_b1, acc, sem1, sem2, sem3, sem4):
        c = lax.axis_index("c")
        s = lax.axis_index("s")
        iota16 = lax.iota(jnp.int32, 16)
        tbase = s * tpt
        zero16 = jnp.zeros((16,), jnp.float32)

        for chunk in range(nchunk):
            base = c * (nchunk * ch) + chunk * ch

            def zr(i, _):
                for cc in range(ie // 16):
                    rows_a0[i, pl.ds(cc * 16, 16)] = zero16
                return 0

            lax.fori_loop(0, _BB, zr, 0)
            _zero_rows(rows_a0, acc, s * share, share)

            @pl.when(s == n_sub - 1)
            def _zero_tail():
                _zero_rows(rows_a0, acc, n_sub * share, zrem - share)

            plsc.subcore_barrier()

            m = jnp.int32(0)
            for w in range(nwin):
                woff = pl.multiple_of(tbase + w * _WSZ, 8)
                pltpu.sync_copy(ji_hbm.at[pl.ds(woff, _WSZ)], win_ji)
                pltpu.sync_copy(kj_hbm.at[pl.ds(woff, _WSZ)], win_kj)

                def grp(g, mm, w=w):
                    ji = win_ji[pl.ds(g * 16, 16)]
                    msk = (ji >= base) & (ji < base + ch)
                    mi = jnp.where(msk, 1, 0)
                    incl = plsc.cumsum(mi)
                    pos = mm + incl - mi
                    td = ((ji - base) << 15) | (w * _WSZ + g * 16 + iota16)
                    plsc.store_scatter(lst_td, [pos], td, mask=msk)
                    kjv = win_kj[pl.ds(g * 16, 16)]
                    plsc.store_scatter(lst_kj, [pos], kjv, mask=msk)
                    return mm + incl[15]

                m = lax.fori_loop(0, wgrp, grp, m)

            nb = (m + (_BB - 1)) // _BB
            nb2 = (nb + 1) // 2

            def prep_fire(bq, buf, ra, rb, s_a, s_b):
                o = bq * _BB
                for g in range(_BB // 16):
                    td = lst_td[pl.ds(o + g * 16, 16)]
                    kj = lst_kj[pl.ds(o + g * 16, 16)]
                    pos = o + g * 16 + iota16
                    pred = pos < m
                    trel = td & jnp.int32(0x7FFF)
                    dstv = lax.shift_right_logical(td, 15)
                    bat_t[buf, pl.ds(g * 16, 16)] = jnp.where(
                        pred, trel + tbase, tbase)
                    bat_kj[buf, pl.ds(g * 16, 16)] = jnp.where(
                        pred, kj, iota16)
                    bat_dst[buf, pl.ds(g * 16, 16)] = jnp.where(
                        pred, dstv, jnp.int32(ch))
                pltpu.async_copy(tmp_hbm.at[bat_kj.at[buf]], ra, s_a)
                pltpu.async_copy(sbft_hbm.at[bat_t.at[buf]], rb, s_b)

            def wait_mul_scatter(buf, ra, rb, s_a, s_b):
                pltpu.make_async_copy(
                    tmp_hbm.at[bat_kj.at[buf]], ra, s_a).wait()
                pltpu.make_async_copy(
                    sbft_hbm.at[bat_t.at[buf]], rb, s_b).wait()

                def mulrow(r, _):
                    for cc in range(ie // 32):
                        sl = pl.ds(cc * 16, 16)
                        ra[r, sl] = ra[r, sl] * rb[r, sl]
                    return 0

                lax.fori_loop(0, _BB, mulrow, 0)
                pltpu.sync_copy(ra, acc.at[bat_dst.at[buf]], add=True)

            prep_fire(jnp.int32(0), 0, rows_a0, rows_b0, sem1, sem2)

            def batch_pair(i, _):
                prep_fire(2 * i + 1, 1, rows_a1, rows_b1, sem3, sem4)
                wait_mul_scatter(0, rows_a0, rows_b0, sem1, sem2)
                prep_fire(2 * i + 2, 0, rows_a0, rows_b0, sem1, sem2)
                wait_mul_scatter(1, rows_a1, rows_b1, sem3, sem4)
                return 0

            lax.fori_loop(0, nb2, batch_pair, 0)
            pltpu.make_async_copy(
                tmp_hbm.at[bat_kj.at[0]], rows_a0, sem1).wait()
            pltpu.make_async_copy(
                sbft_hbm.at[bat_t.at[0]], rows_b0, sem2).wait()
            plsc.subcore_barrier()

            pltpu.sync_copy(acc.at[pl.ds(s * share, share)],
                            out_hbm.at[pl.ds(base + s * share, share)])

            if wrem > share:
                @pl.when(s == n_sub - 1)
                def _write_tail():
                    tail = wrem - share
                    off = n_sub * share
                    pltpu.sync_copy(acc.at[pl.ds(off, tail)],
                                    out_hbm.at[pl.ds(base + off, tail)])

            plsc.subcore_barrier()

    return k(tmp, sbft, idx_kj, idx_ji)


def kernel(x, rbf, sbf, idx_kj, idx_ji, bt, lambda_d, alpha,
           W_rbf1, W_rbf2, W_sbf1, W_sbf2, W_kj, b_kj, W_ji, b_ji,
           W_down, W_up, W_b1, b_b1, W_b2, b_b2, W_lin, b_lin,
           W_a1, b_a1, W_a2, b_a2):
    nbt = W_kj.shape[0]
    b = nbt - 1
    h = x.shape[1]
    blk = 1600

    tmp = _edge_mlp(x, rbf, W_kj[b], b_kj[b].reshape(1, h),
                    W_rbf1[b], W_rbf2[b], W_down[b], blk)
    sbft = _sbf_mlp(sbf, W_sbf1[b], W_sbf2[b], blk)
    xkt = _sc_segsum(tmp, sbft, idx_kj.astype(jnp.int32),
                     idx_ji.astype(jnp.int32))
    w_up_eff = jnp.concatenate(
        [W_up * jnp.asarray(alpha, jnp.float32), jnp.zeros_like(W_up)],
        axis=0)
    return _tail_mlp(xkt, x, w_up_eff, W_ji, b_ji.reshape(1, h),
                     W_b1, b_b1.reshape(1, h), W_b2, b_b2.reshape(1, h),
                     W_lin, b_lin.reshape(1, h), W_a1, b_a1.reshape(1, h),
                     W_a2, b_a2.reshape(1, h), blk)
